# baseline (speedup 1.0000x reference)
import functools

import jax
import jax.numpy as jnp
from jax import lax
from jax.experimental import pallas as pl
from jax.experimental.pallas import tpu as pltpu
from jax.experimental.pallas import tpu_sc as plsc

_TAU_NUM_STEPS = 50
_NUM_TRAIN_STEPS = 1000
_TAB_PAD = 64
_CHUNK = 16


def _tau_tables():
    betas = jnp.linspace(1e-4, 0.02, _NUM_TRAIN_STEPS, dtype=jnp.float32)
    alphas = 1.0 - betas
    alpha_bars = jnp.cumprod(alphas)
    tau = jnp.linspace(0, _NUM_TRAIN_STEPS - 1, _TAU_NUM_STEPS).astype(jnp.int32)
    tau_alpha_bars = jnp.take(alpha_bars, tau)
    sac = jnp.sqrt(tau_alpha_bars)
    somac = jnp.sqrt(1.0 - tau_alpha_bars)
    pad = _TAB_PAD - _TAU_NUM_STEPS
    rsac = 1.0 / sac
    dtab = -rsac * somac
    one = jnp.ones((pad,), jnp.float32)
    return jnp.stack([
        jnp.concatenate([rsac, one]),
        jnp.concatenate([dtab, one]),
        jnp.concatenate([sac, one]),
        jnp.concatenate([somac, one]),
    ])


def _zero16():
    return lax.iota(jnp.int32, _CHUNK) * 0


def _sc_coeff_body(tabs_h, ts_h, pts_h, out_h, tabs_v, ts_v, pts_v, coef_v):
    info = plsc.get_sparse_core_info()
    wid = lax.axis_index("s") * info.num_cores + lax.axis_index("c")
    nchunks = 64 // _CHUNK

    @pl.when(wid < nchunks)
    def _():
        base = wid * _CHUNK
        pltpu.sync_copy(tabs_h, tabs_v)
        pltpu.sync_copy(ts_h.at[pl.ds(base, _CHUNK)], ts_v)
        pltpu.sync_copy(pts_h.at[pl.ds(base, _CHUNK)], pts_v)
        t = ts_v[...]
        p = pts_v[...]
        z = _zero16()
        c = plsc.load_gather(tabs_v, [z, t])
        d = plsc.load_gather(tabs_v, [z + 1, t])
        psac = plsc.load_gather(tabs_v, [z + 2, p])
        psomac = plsc.load_gather(tabs_v, [z + 3, p])
        coef_v[0] = psac * c
        coef_v[1] = psac * d + psomac
        coef_v[2] = c
        coef_v[3] = d
        pltpu.sync_copy(coef_v, out_h.at[wid])


def _sc_coeffs(tabs, ts, pts):
    mesh = plsc.VectorSubcoreMesh(core_axis_name="c", subcore_axis_name="s")
    run = functools.partial(
        pl.kernel,
        mesh=mesh,
        compiler_params=pltpu.CompilerParams(needs_layout_passes=False),
        out_type=jax.ShapeDtypeStruct((4, 4, _CHUNK), jnp.float32),
        scratch_types=[
            pltpu.VMEM((4, _TAB_PAD), jnp.float32),
            pltpu.VMEM((_CHUNK,), jnp.int32),
            pltpu.VMEM((_CHUNK,), jnp.int32),
            pltpu.VMEM((4, _CHUNK), jnp.float32),
        ],
    )(_sc_coeff_body)
    return run(tabs, ts, pts)


def _tc_body(coef_ref, xt_ref, pn_ref, xtp_ref, x0_ref):
    i = pl.program_id(0)
    g = xt_ref.shape[0]
    for k in range(g):
        s = i * g + k
        chunk = s // _CHUNK
        lane = s % _CHUNK
        a = coef_ref[chunk, 0, lane]
        b = coef_ref[chunk, 1, lane]
        c = coef_ref[chunk, 2, lane]
        d = coef_ref[chunk, 3, lane]
        xt_v = xt_ref[k]
        pn = pn_ref[k]
        x0_ref[k] = c * xt_v + d * pn
        xtp_ref[k] = a * xt_v + b * pn


_GROUP = 8


def kernel(xt, predicted_noise, time_steps, prev_time_steps):
    B, C, H, W = xt.shape
    rows = C * H * W // W
    tabs = _tau_tables()
    coeffs = _sc_coeffs(
        tabs, time_steps.astype(jnp.int32), prev_time_steps.astype(jnp.int32)
    )
    x3 = xt.reshape(B, rows, W)
    p3 = predicted_noise.reshape(B, rows, W)
    g = _GROUP
    grid_spec = pltpu.PrefetchScalarGridSpec(
        num_scalar_prefetch=1,
        grid=(B // g,),
        in_specs=[
            pl.BlockSpec((g, rows, W), lambda i, *_: (i, 0, 0)),
            pl.BlockSpec((g, rows, W), lambda i, *_: (i, 0, 0)),
        ],
        out_specs=[
            pl.BlockSpec((g, rows, W), lambda i, *_: (i, 0, 0)),
            pl.BlockSpec((g, rows, W), lambda i, *_: (i, 0, 0)),
        ],
    )
    xtp, x0 = pl.pallas_call(
        _tc_body,
        grid_spec=grid_spec,
        out_shape=[jax.ShapeDtypeStruct((B, rows, W), jnp.float32)] * 2,
        compiler_params=pltpu.CompilerParams(
            dimension_semantics=("parallel",),
        ),
    )(coeffs, x3, p3)
    return xtp.reshape(B, C, H, W), x0.reshape(B, C, H, W)


# speedup vs baseline: 1.0003x; 1.0003x over previous
import functools

import jax
import jax.numpy as jnp
import numpy as np
from jax import lax
from jax.experimental import pallas as pl
from jax.experimental.pallas import tpu as pltpu
from jax.experimental.pallas import tpu_sc as plsc

_TAU_NUM_STEPS = 50
_NUM_TRAIN_STEPS = 1000
_TAB_PAD = 64
_CHUNK = 16


def _tau_tables():
    betas = np.linspace(1e-4, 0.02, _NUM_TRAIN_STEPS, dtype=np.float32)
    alphas = (1.0 - betas).astype(np.float32)
    alpha_bars = np.cumprod(alphas, dtype=np.float32)
    tau = np.linspace(0, _NUM_TRAIN_STEPS - 1, _TAU_NUM_STEPS).astype(np.int32)
    tab = alpha_bars[tau]
    sac = np.sqrt(tab).astype(np.float32)
    somac = np.sqrt((1.0 - tab).astype(np.float32)).astype(np.float32)
    rsac = (1.0 / sac).astype(np.float32)
    dtab = (-rsac * somac).astype(np.float32)
    out = np.ones((4, _TAB_PAD), np.float32)
    out[0, :_TAU_NUM_STEPS] = rsac
    out[1, :_TAU_NUM_STEPS] = dtab
    out[2, :_TAU_NUM_STEPS] = sac
    out[3, :_TAU_NUM_STEPS] = somac
    return out


_TABS = _tau_tables()


def _zero16():
    return lax.iota(jnp.int32, _CHUNK) * 0


def _sc_coeff_body(tabs_h, ts_h, pts_h, out_h, tabs_v, ts_v, pts_v, coef_v):
    info = plsc.get_sparse_core_info()
    wid = lax.axis_index("s") * info.num_cores + lax.axis_index("c")
    nchunks = 64 // _CHUNK

    @pl.when(wid < nchunks)
    def _():
        base = wid * _CHUNK
        pltpu.sync_copy(tabs_h, tabs_v)
        pltpu.sync_copy(ts_h.at[pl.ds(base, _CHUNK)], ts_v)
        pltpu.sync_copy(pts_h.at[pl.ds(base, _CHUNK)], pts_v)
        t = ts_v[...]
        p = pts_v[...]
        z = _zero16()
        c = plsc.load_gather(tabs_v, [z, t])
        d = plsc.load_gather(tabs_v, [z + 1, t])
        psac = plsc.load_gather(tabs_v, [z + 2, p])
        psomac = plsc.load_gather(tabs_v, [z + 3, p])
        coef_v[0] = psac * c
        coef_v[1] = psac * d + psomac
        coef_v[2] = c
        coef_v[3] = d
        pltpu.sync_copy(coef_v, out_h.at[wid])


def _sc_coeffs(tabs, ts, pts):
    mesh = plsc.VectorSubcoreMesh(core_axis_name="c", subcore_axis_name="s")
    run = functools.partial(
        pl.kernel,
        mesh=mesh,
        compiler_params=pltpu.CompilerParams(needs_layout_passes=False),
        out_type=jax.ShapeDtypeStruct((4, 4, _CHUNK), jnp.float32),
        scratch_types=[
            pltpu.VMEM((4, _TAB_PAD), jnp.float32),
            pltpu.VMEM((_CHUNK,), jnp.int32),
            pltpu.VMEM((_CHUNK,), jnp.int32),
            pltpu.VMEM((4, _CHUNK), jnp.float32),
        ],
    )(_sc_coeff_body)
    return run(tabs, ts, pts)


def _tc_body(coef_ref, xt_ref, pn_ref, xtp_ref, x0_ref):
    i = pl.program_id(0)
    g = xt_ref.shape[0]
    for k in range(g):
        s = i * g + k
        chunk = s // _CHUNK
        lane = s % _CHUNK
        a = coef_ref[chunk, 0, lane]
        b = coef_ref[chunk, 1, lane]
        c = coef_ref[chunk, 2, lane]
        d = coef_ref[chunk, 3, lane]
        xt_v = xt_ref[k]
        pn = pn_ref[k]
        x0_ref[k] = c * xt_v + d * pn
        xtp_ref[k] = a * xt_v + b * pn


_GROUP = 8


def kernel(xt, predicted_noise, time_steps, prev_time_steps):
    B, C, H, W = xt.shape
    rows = C * H * W // W
    tabs = jnp.asarray(_TABS)
    coeffs = _sc_coeffs(
        tabs, time_steps.astype(jnp.int32), prev_time_steps.astype(jnp.int32)
    )
    x3 = xt.reshape(B, rows, W)
    p3 = predicted_noise.reshape(B, rows, W)
    g = _GROUP
    grid_spec = pltpu.PrefetchScalarGridSpec(
        num_scalar_prefetch=1,
        grid=(B // g,),
        in_specs=[
            pl.BlockSpec((g, rows, W), lambda i, *_: (i, 0, 0)),
            pl.BlockSpec((g, rows, W), lambda i, *_: (i, 0, 0)),
        ],
        out_specs=[
            pl.BlockSpec((g, rows, W), lambda i, *_: (i, 0, 0)),
            pl.BlockSpec((g, rows, W), lambda i, *_: (i, 0, 0)),
        ],
    )
    xtp, x0 = pl.pallas_call(
        _tc_body,
        grid_spec=grid_spec,
        out_shape=[jax.ShapeDtypeStruct((B, rows, W), jnp.float32)] * 2,
        compiler_params=pltpu.CompilerParams(
            dimension_semantics=("parallel",),
        ),
    )(coeffs, x3, p3)
    return xtp.reshape(B, C, H, W), x0.reshape(B, C, H, W)


# speedup vs baseline: 1.0227x; 1.0223x over previous
import functools

import jax
import jax.numpy as jnp
import numpy as np
from jax import lax
from jax.experimental import pallas as pl
from jax.experimental.pallas import tpu as pltpu
from jax.experimental.pallas import tpu_sc as plsc

_TAU_NUM_STEPS = 50
_NUM_TRAIN_STEPS = 1000
_TAB_PAD = 64
_CHUNK = 16


def _tau_tables():
    betas = np.linspace(1e-4, 0.02, _NUM_TRAIN_STEPS, dtype=np.float32)
    alphas = (1.0 - betas).astype(np.float32)
    alpha_bars = np.cumprod(alphas, dtype=np.float32)
    tau = np.linspace(0, _NUM_TRAIN_STEPS - 1, _TAU_NUM_STEPS).astype(np.int32)
    tab = alpha_bars[tau]
    sac = np.sqrt(tab).astype(np.float32)
    somac = np.sqrt((1.0 - tab).astype(np.float32)).astype(np.float32)
    rsac = (1.0 / sac).astype(np.float32)
    dtab = (-rsac * somac).astype(np.float32)
    out = np.ones((4, _TAB_PAD), np.float32)
    out[0, :_TAU_NUM_STEPS] = rsac
    out[1, :_TAU_NUM_STEPS] = dtab
    out[2, :_TAU_NUM_STEPS] = sac
    out[3, :_TAU_NUM_STEPS] = somac
    return out


_TABS = _tau_tables()


def _zero16():
    return lax.iota(jnp.int32, _CHUNK) * 0


def _sc_coeff_body(tabs_h, ts_h, pts_h, out_h, tabs_v, ts_v, pts_v, coef_v):
    wid = lax.axis_index("s") + lax.axis_index("c")
    nchunks = 64 // _CHUNK

    @pl.when(wid < nchunks)
    def _():
        base = wid * _CHUNK
        pltpu.sync_copy(tabs_h, tabs_v)
        pltpu.sync_copy(ts_h.at[pl.ds(base, _CHUNK)], ts_v)
        pltpu.sync_copy(pts_h.at[pl.ds(base, _CHUNK)], pts_v)
        t = ts_v[...]
        p = pts_v[...]
        z = _zero16()
        c = plsc.load_gather(tabs_v, [z, t])
        d = plsc.load_gather(tabs_v, [z + 1, t])
        psac = plsc.load_gather(tabs_v, [z + 2, p])
        psomac = plsc.load_gather(tabs_v, [z + 3, p])
        coef_v[0] = psac * c
        coef_v[1] = psac * d + psomac
        coef_v[2] = c
        coef_v[3] = d
        pltpu.sync_copy(coef_v, out_h.at[wid])


def _sc_coeffs(tabs, ts, pts):
    mesh = plsc.VectorSubcoreMesh(
        core_axis_name="c", subcore_axis_name="s", num_cores=1
    )
    run = functools.partial(
        pl.kernel,
        mesh=mesh,
        compiler_params=pltpu.CompilerParams(needs_layout_passes=False),
        out_type=jax.ShapeDtypeStruct((4, 4, _CHUNK), jnp.float32),
        scratch_types=[
            pltpu.VMEM((4, _TAB_PAD), jnp.float32),
            pltpu.VMEM((_CHUNK,), jnp.int32),
            pltpu.VMEM((_CHUNK,), jnp.int32),
            pltpu.VMEM((4, _CHUNK), jnp.float32),
        ],
    )(_sc_coeff_body)
    return run(tabs, ts, pts)


def _tc_body(coef_ref, xt_ref, pn_ref, xtp_ref, x0_ref):
    i = pl.program_id(0)
    g = xt_ref.shape[0]
    for k in range(g):
        s = i * g + k
        chunk = s // _CHUNK
        lane = s % _CHUNK
        a = coef_ref[chunk, 0, lane]
        b = coef_ref[chunk, 1, lane]
        c = coef_ref[chunk, 2, lane]
        d = coef_ref[chunk, 3, lane]
        xt_v = xt_ref[k]
        pn = pn_ref[k]
        x0_ref[k] = c * xt_v + d * pn
        xtp_ref[k] = a * xt_v + b * pn


_GROUP = 8


def kernel(xt, predicted_noise, time_steps, prev_time_steps):
    B, C, H, W = xt.shape
    rows = C * H * W // W
    tabs = jnp.asarray(_TABS)
    coeffs = _sc_coeffs(
        tabs, time_steps.astype(jnp.int32), prev_time_steps.astype(jnp.int32)
    )
    x3 = xt.reshape(B, rows, W)
    p3 = predicted_noise.reshape(B, rows, W)
    g = _GROUP
    grid_spec = pltpu.PrefetchScalarGridSpec(
        num_scalar_prefetch=1,
        grid=(B // g,),
        in_specs=[
            pl.BlockSpec((g, rows, W), lambda i, *_: (i, 0, 0)),
            pl.BlockSpec((g, rows, W), lambda i, *_: (i, 0, 0)),
        ],
        out_specs=[
            pl.BlockSpec((g, rows, W), lambda i, *_: (i, 0, 0)),
            pl.BlockSpec((g, rows, W), lambda i, *_: (i, 0, 0)),
        ],
    )
    xtp, x0 = pl.pallas_call(
        _tc_body,
        grid_spec=grid_spec,
        out_shape=[jax.ShapeDtypeStruct((B, rows, W), jnp.float32)] * 2,
        compiler_params=pltpu.CompilerParams(
            dimension_semantics=("parallel",),
        ),
    )(coeffs, x3, p3)
    return xtp.reshape(B, C, H, W), x0.reshape(B, C, H, W)
